# baseline (speedup 1.0000x reference)
import functools

import jax
import jax.numpy as jnp
from jax import lax
from jax.experimental import pallas as pl
from jax.experimental.pallas import tpu as pltpu
from jax.experimental.pallas import tpu_sc as plsc

N = 10000
E = 320000
NP = 10112
W = 40
NT = 32
CH = 80
EP = NT * CH * 128
ROWS = NP // 16
NB = 8

_mesh = plsc.VectorSubcoreMesh(core_axis_name="c", subcore_axis_name="s")
_sc_params = pltpu.CompilerParams(needs_layout_passes=False,
                                  use_tc_tiling_on_sc=False)


@functools.partial(
    pl.kernel,
    out_type=jax.ShapeDtypeStruct((NT, 2, NP), jnp.float32),
    mesh=_mesh,
    compiler_params=_sc_params,
    scratch_types=[
        pltpu.VMEM((CH, 128), jnp.int32),
        pltpu.VMEM((CH, 128), jnp.int32),
        pltpu.VMEM((NP,), jnp.float32),
        pltpu.VMEM((NP,), jnp.float32),
    ],
)
def _deg_kernel(src_hbm, dst_hbm, out_hbm, src_v, dst_v, hs_v, hd_v):
    c = lax.axis_index("c")
    s = lax.axis_index("s")
    wid = c * 16 + s
    pltpu.sync_copy(src_hbm.at[wid], src_v)
    pltpu.sync_copy(dst_hbm.at[wid], dst_v)
    zeros = jnp.zeros((16,), jnp.float32)

    @pl.loop(0, NP // 16)
    def _(i):
        hs_v[pl.ds(i * 16, 16)] = zeros
        hd_v[pl.ds(i * 16, 16)] = zeros

    ones = jnp.ones((16,), jnp.float32)

    @pl.loop(0, CH)
    def _(k):
        for j in range(8):
            si = src_v[k, pl.ds(j * 16, 16)]
            plsc.addupdate_scatter(hs_v, [si], ones)
            di = dst_v[k, pl.ds(j * 16, 16)]
            plsc.addupdate_scatter(hd_v, [di], ones)

    pltpu.sync_copy(hs_v, out_hbm.at[wid, 0])
    pltpu.sync_copy(hd_v, out_hbm.at[wid, 1])


@functools.partial(
    pl.kernel,
    out_type=jax.ShapeDtypeStruct((2, NP, W), jnp.float32),
    mesh=_mesh,
    compiler_params=_sc_params,
    scratch_types=[
        pltpu.VMEM((CH, 128), jnp.int32),
        pltpu.VMEM((CH, 128), jnp.int32),
        pltpu.VMEM((NB, 128, W), jnp.float32),
        pltpu.VMEM_SHARED((NP, W), jnp.float32),
        pltpu.SemaphoreType.DMA((NB,)),
        pltpu.SemaphoreType.DMA((NB,)),
    ],
)
def _spmv_kernel(u_hbm, src_hbm, dst_hbm, zer_hbm, z_hbm,
                 src_v, dst_v, bufs, acc, gsem, ssem):
    c = lax.axis_index("c")
    s = lax.axis_index("s")
    wid = c * 16 + s
    pltpu.sync_copy(src_hbm.at[wid], src_v)
    pltpu.sync_copy(dst_hbm.at[wid], dst_v)
    pltpu.sync_copy(zer_hbm.at[pl.ds(s * ROWS, ROWS)],
                    acc.at[pl.ds(s * ROWS, ROWS)])
    plsc.subcore_barrier()

    for j in range(NB):
        pltpu.async_copy(u_hbm.at[src_v.at[j]], bufs.at[j], gsem.at[j])

    @pl.loop(0, CH, step=NB)
    def _(k):
        for j in range(NB):
            kk = k + j
            pltpu.make_async_copy(u_hbm.at[src_v.at[kk]], bufs.at[j],
                                  gsem.at[j]).wait()
            pltpu.async_copy(bufs.at[j], acc.at[dst_v.at[kk]], ssem.at[j],
                             add=True)
        for j in range(NB):
            kk = k + j
            pltpu.make_async_copy(bufs.at[j], acc.at[dst_v.at[kk]],
                                  ssem.at[j]).wait()

            @pl.when(kk + NB < CH)
            def _():
                pltpu.async_copy(u_hbm.at[src_v.at[kk + NB]], bufs.at[j],
                                 gsem.at[j])

    plsc.subcore_barrier()
    pltpu.sync_copy(acc.at[pl.ds(s * ROWS, ROWS)],
                    z_hbm.at[c, pl.ds(s * ROWS, ROWS)])


NBLK = 8
BR = NP // NBLK


def _mm_body(feat_ref, w0, w1, w2, l0, l1, l2, y0, y1, y2, y3):
    W0, W1, W2 = w0[...], w1[...], w2[...]
    L0, L1, L2 = l0[...], l1[...], l2[...]
    C3 = W0 @ W1 @ W2
    C2 = L0 @ W1 @ W2 + W0 @ L1 @ W2 + W0 @ W1 @ L2
    C1 = L0 @ L1 @ W2 + L0 @ W1 @ L2 + W0 @ L1 @ L2
    C0 = L0 @ L1 @ L2
    X = feat_ref[...]
    y0[...] = X @ C0
    y1[...] = X @ C1
    y2[...] = X @ C2
    y3[...] = X @ C3


def _wspec(shape):
    return pl.BlockSpec(shape, lambda i: (0, 0))


_mm_call = pl.pallas_call(
    _mm_body,
    grid=(NBLK,),
    in_specs=[
        pl.BlockSpec((BR, 128), lambda i: (i, 0)),
        _wspec((128, 128)), _wspec((128, 128)), _wspec((128, 40)),
        _wspec((128, 128)), _wspec((128, 128)), _wspec((128, 40)),
    ],
    out_specs=[pl.BlockSpec((BR, W), lambda i: (i, 0))] * 4,
    out_shape=[jax.ShapeDtypeStruct((NP, W), jnp.float32)] * 4,
)


def _scale_body(y0r, y1r, y2r, y3r, dsrc_ref, ddst_ref, b2r,
                y0b, yh1, yh2, u3, nsdo, nddo):
    deg_s = jnp.sum(dsrc_ref[...], axis=1, keepdims=True)
    deg_d = jnp.sum(ddst_ref[...], axis=1, keepdims=True)
    ns = lax.rsqrt(jnp.maximum(deg_s, 1.0))
    nd = lax.rsqrt(jnp.maximum(deg_d, 1.0))
    y0b[...] = y0r[...] + b2r[...]
    yh1[...] = ns * y1r[...]
    yh2[...] = ns * y2r[...]
    u3[...] = ns * y3r[...]
    nsdo[...] = ns * nd
    nddo[...] = nd


_scale_call = pl.pallas_call(
    _scale_body,
    grid=(NBLK,),
    in_specs=[
        pl.BlockSpec((BR, W), lambda i: (i, 0)),
        pl.BlockSpec((BR, W), lambda i: (i, 0)),
        pl.BlockSpec((BR, W), lambda i: (i, 0)),
        pl.BlockSpec((BR, W), lambda i: (i, 0)),
        pl.BlockSpec((BR, NT), lambda i: (i, 0)),
        pl.BlockSpec((BR, NT), lambda i: (i, 0)),
        _wspec((1, W)),
    ],
    out_specs=[
        pl.BlockSpec((BR, W), lambda i: (i, 0)),
        pl.BlockSpec((BR, W), lambda i: (i, 0)),
        pl.BlockSpec((BR, W), lambda i: (i, 0)),
        pl.BlockSpec((BR, W), lambda i: (i, 0)),
        pl.BlockSpec((BR, 1), lambda i: (i, 0)),
        pl.BlockSpec((BR, 1), lambda i: (i, 0)),
    ],
    out_shape=[
        jax.ShapeDtypeStruct((NP, W), jnp.float32),
        jax.ShapeDtypeStruct((NP, W), jnp.float32),
        jax.ShapeDtypeStruct((NP, W), jnp.float32),
        jax.ShapeDtypeStruct((NP, W), jnp.float32),
        jax.ShapeDtypeStruct((NP, 1), jnp.float32),
        jax.ShapeDtypeStruct((NP, 1), jnp.float32),
    ],
)


def _comb_body(z_ref, yh_ref, sc_ref, u_ref):
    u_ref[...] = yh_ref[...] + sc_ref[...] * (z_ref[0] + z_ref[1])


_comb_call = pl.pallas_call(
    _comb_body,
    out_shape=jax.ShapeDtypeStruct((NP, W), jnp.float32),
)


def kernel(feat, edge_index, W0, W1, W2, b2, L0, L1, L2):
    src = edge_index[0]
    dst = edge_index[1]
    pad = EP - E
    fill = N + (jnp.arange(pad, dtype=jnp.int32) % (NP - N))
    srcp = jnp.concatenate([src, fill]).reshape(NT, CH, 128)
    dstp = jnp.concatenate([dst, fill]).reshape(NT, CH, 128)
    featp = jnp.pad(feat, ((0, NP - N), (0, 0)))
    b2p = jnp.pad(b2, (0, W - 40)).reshape(1, W)
    zer = jnp.zeros((NP, W), jnp.float32)

    degp = _deg_kernel(srcp, dstp)
    dsrc_t = jnp.transpose(degp[:, 0, :])
    ddst_t = jnp.transpose(degp[:, 1, :])
    y0, y1, y2, y3 = _mm_call(featp, W0, W1, W2, L0, L1, L2)
    y0b, yh1, yh2, u3, nsd, ndc = _scale_call(y0, y1, y2, y3,
                                              dsrc_t, ddst_t, b2p)
    z3 = _spmv_kernel(u3, srcp, dstp, zer)
    u2 = _comb_call(z3, yh2, nsd)
    z2 = _spmv_kernel(u2, srcp, dstp, zer)
    u1 = _comb_call(z2, yh1, nsd)
    z1 = _spmv_kernel(u1, srcp, dstp, zer)
    outp = _comb_call(z1, y0b, ndc)
    return outp[:N, :40]


# speedup vs baseline: 1.1738x; 1.1738x over previous
import functools

import jax
import jax.numpy as jnp
from jax import lax
from jax.experimental import pallas as pl
from jax.experimental.pallas import tpu as pltpu
from jax.experimental.pallas import tpu_sc as plsc

N = 10000
E = 320000
NP = 10112
W = 40
NT = 32
CH = 80
EP = NT * CH * 128
ROWS = NP // 16
NB = 8

_mesh = plsc.VectorSubcoreMesh(core_axis_name="c", subcore_axis_name="s")
_sc_params = pltpu.CompilerParams(needs_layout_passes=False,
                                  use_tc_tiling_on_sc=False)


@functools.partial(
    pl.kernel,
    out_type=jax.ShapeDtypeStruct((NT, 2, NP), jnp.float32),
    mesh=_mesh,
    compiler_params=_sc_params,
    scratch_types=[
        pltpu.VMEM((CH, 128), jnp.int32),
        pltpu.VMEM((CH, 128), jnp.int32),
        pltpu.VMEM((NP,), jnp.float32),
        pltpu.VMEM((NP,), jnp.float32),
    ],
)
def _deg_kernel(src_hbm, dst_hbm, out_hbm, src_v, dst_v, hs_v, hd_v):
    c = lax.axis_index("c")
    s = lax.axis_index("s")
    wid = c * 16 + s
    pltpu.sync_copy(src_hbm.at[wid], src_v)
    pltpu.sync_copy(dst_hbm.at[wid], dst_v)
    zeros = jnp.zeros((16,), jnp.float32)

    @pl.loop(0, NP // 16)
    def _(i):
        hs_v[pl.ds(i * 16, 16)] = zeros
        hd_v[pl.ds(i * 16, 16)] = zeros

    ones = jnp.ones((16,), jnp.float32)

    @pl.loop(0, CH)
    def _(k):
        for j in range(8):
            si = src_v[k, pl.ds(j * 16, 16)]
            plsc.addupdate_scatter(hs_v, [si], ones)
            di = dst_v[k, pl.ds(j * 16, 16)]
            plsc.addupdate_scatter(hd_v, [di], ones)

    pltpu.sync_copy(hs_v, out_hbm.at[wid, 0])
    pltpu.sync_copy(hd_v, out_hbm.at[wid, 1])


@functools.partial(
    pl.kernel,
    out_type=jax.ShapeDtypeStruct((2, NP, W), jnp.float32),
    mesh=_mesh,
    compiler_params=_sc_params,
    scratch_types=[
        pltpu.VMEM((CH, 128), jnp.int32),
        pltpu.VMEM((CH, 128), jnp.int32),
        pltpu.VMEM((NB, 128, W), jnp.float32),
        pltpu.VMEM_SHARED((NP, W), jnp.float32),
        pltpu.SemaphoreType.DMA((NB,)),
        pltpu.SemaphoreType.DMA((NB,)),
    ],
)
def _spmv_kernel(u_hbm, src_hbm, dst_hbm, zer_hbm, z_hbm,
                 src_v, dst_v, bufs, acc, gsem, ssem):
    c = lax.axis_index("c")
    s = lax.axis_index("s")
    wid = c * 16 + s
    pltpu.sync_copy(src_hbm.at[wid], src_v)
    pltpu.sync_copy(dst_hbm.at[wid], dst_v)
    pltpu.sync_copy(zer_hbm.at[pl.ds(s * ROWS, ROWS)],
                    acc.at[pl.ds(s * ROWS, ROWS)])
    plsc.subcore_barrier()

    for j in range(NB):
        pltpu.async_copy(u_hbm.at[src_v.at[j]], bufs.at[j], gsem.at[j])

    @pl.loop(0, CH, step=NB)
    def _(k):
        for j in range(NB):
            kk = k + j
            pltpu.make_async_copy(u_hbm.at[src_v.at[kk]], bufs.at[j],
                                  gsem.at[j]).wait()
            pltpu.async_copy(bufs.at[j], acc.at[dst_v.at[kk]], ssem.at[j],
                             add=True)
        for j in range(NB):
            kk = k + j
            pltpu.make_async_copy(bufs.at[j], acc.at[dst_v.at[kk]],
                                  ssem.at[j]).wait()

            @pl.when(kk + NB < CH)
            def _():
                pltpu.async_copy(u_hbm.at[src_v.at[kk + NB]], bufs.at[j],
                                 gsem.at[j])

    plsc.subcore_barrier()
    pltpu.sync_copy(acc.at[pl.ds(s * ROWS, ROWS)],
                    z_hbm.at[c, pl.ds(s * ROWS, ROWS)])


NBLK = 8
BR = NP // NBLK
FB = BR * W


def _dense_body(feat_ref, dsrc_ref, ddst_ref, w0, w1, w2, b2r, l0, l1, l2,
                y0b, yh1, yh2, u3, nsdo, nddo):
    W0, W1, W2 = w0[...], w1[...], w2[...]
    L0, L1, L2 = l0[...], l1[...], l2[...]
    C3 = W0 @ W1 @ W2
    C2 = L0 @ W1 @ W2 + W0 @ L1 @ W2 + W0 @ W1 @ L2
    C1 = L0 @ L1 @ W2 + L0 @ W1 @ L2 + W0 @ L1 @ L2
    C0 = L0 @ L1 @ L2

    X = feat_ref[...]
    deg_s = jnp.sum(dsrc_ref[...], axis=1, keepdims=True)
    deg_d = jnp.sum(ddst_ref[...], axis=1, keepdims=True)
    ns = lax.rsqrt(jnp.maximum(deg_s, 1.0))
    nd = lax.rsqrt(jnp.maximum(deg_d, 1.0))
    y0b[...] = X @ C0 + b2r[...]
    yh1[...] = ns * (X @ C1)
    yh2[...] = ns * (X @ C2)
    u3[...] = ns * (X @ C3)
    nsdo[...] = jnp.broadcast_to(ns * nd, (BR, W))
    nddo[...] = jnp.broadcast_to(nd, (BR, W))


def _wspec(shape):
    return pl.BlockSpec(shape, lambda i: (0, 0))


_dense_call = pl.pallas_call(
    _dense_body,
    grid=(NBLK,),
    in_specs=[
        pl.BlockSpec((BR, 128), lambda i: (i, 0)),
        pl.BlockSpec((BR, NT), lambda i: (i, 0)),
        pl.BlockSpec((BR, NT), lambda i: (i, 0)),
        _wspec((128, 128)), _wspec((128, 128)), _wspec((128, 40)),
        _wspec((1, W)),
        _wspec((128, 128)), _wspec((128, 128)), _wspec((128, 40)),
    ],
    out_specs=[pl.BlockSpec((BR, W), lambda i: (i, 0))] * 6,
    out_shape=[jax.ShapeDtypeStruct((NP, W), jnp.float32)] * 6,
)


def _comb_body(z_ref, yh_ref, sc_ref, u_ref):
    u_ref[...] = yh_ref[...] + sc_ref[...] * (z_ref[0] + z_ref[1])


_comb_call = pl.pallas_call(
    _comb_body,
    out_shape=jax.ShapeDtypeStruct((NP * W,), jnp.float32),
)


def kernel(feat, edge_index, W0, W1, W2, b2, L0, L1, L2):
    src = edge_index[0]
    dst = edge_index[1]
    pad = EP - E
    fill = N + (jnp.arange(pad, dtype=jnp.int32) % (NP - N))
    srcp = jnp.concatenate([src, fill]).reshape(NT, CH, 128)
    dstp = jnp.concatenate([dst, fill]).reshape(NT, CH, 128)
    featp = jnp.pad(feat, ((0, NP - N), (0, 0)))
    b2p = b2.reshape(1, W)
    zer = jnp.zeros((NP, W), jnp.float32)

    degp = _deg_kernel(srcp, dstp)
    dsrc_t = jnp.transpose(degp[:, 0, :])
    ddst_t = jnp.transpose(degp[:, 1, :])
    y0b, yh1, yh2, u3, nsdb, ndb = _dense_call(featp, dsrc_t, ddst_t,
                                               W0, W1, W2, b2p, L0, L1, L2)
    yh2f = yh2.reshape(NP * W)
    yh1f = yh1.reshape(NP * W)
    y0bf = y0b.reshape(NP * W)
    nsf = nsdb.reshape(NP * W)
    ndf = ndb.reshape(NP * W)
    z3 = _spmv_kernel(u3, srcp, dstp, zer)
    u2 = _comb_call(z3.reshape(2, NP * W), yh2f, nsf)
    z2 = _spmv_kernel(u2.reshape(NP, W), srcp, dstp, zer)
    u1 = _comb_call(z2.reshape(2, NP * W), yh1f, nsf)
    z1 = _spmv_kernel(u1.reshape(NP, W), srcp, dstp, zer)
    outp = _comb_call(z1.reshape(2, NP * W), y0bf, ndf)
    return outp.reshape(NP, W)[:N, :40]
